# Initial kernel scaffold; baseline (speedup 1.0000x reference)
import jax
import jax.numpy as jnp
from jax.experimental import pallas as pl


def kernel(x, combos):
    raise NotImplementedError("write your pallas kernel here")


# speedup vs baseline: 7724.6259x; 7724.6259x over previous
import jax
import jax.numpy as jnp
from jax.experimental import pallas as pl

_N_TERMS = 6
_N_MF = 5
_N_RULES = _N_TERMS ** _N_MF
_BB = 128


def _fuzzy_block_kernel(xt_ref, ct_ref, out_ref):
    iota = jax.lax.broadcasted_iota(jnp.int32, (_N_TERMS, _N_RULES), 0)
    acc = None
    for m in range(_N_MF):
        onehot = (ct_ref[m : m + 1, :] == iota).astype(jnp.float32)
        g = jnp.dot(xt_ref[m], onehot, preferred_element_type=jnp.float32)
        acc = g if acc is None else acc * g
    denom = jnp.maximum(jnp.sum(jnp.abs(acc), axis=1, keepdims=True), 1e-12)
    out_ref[...] = acc / denom


def kernel(x, combos):
    b = x.shape[0]
    xt = jnp.transpose(x, (2, 0, 1))
    ct = jnp.transpose(combos, (1, 0))
    grid = b // _BB
    return pl.pallas_call(
        _fuzzy_block_kernel,
        grid=(grid,),
        in_specs=[
            pl.BlockSpec((_N_MF, _BB, _N_TERMS), lambda i: (0, i, 0)),
            pl.BlockSpec((_N_MF, _N_RULES), lambda i: (0, 0)),
        ],
        out_specs=pl.BlockSpec((_BB, _N_RULES), lambda i: (i, 0)),
        out_shape=jax.ShapeDtypeStruct((b, _N_RULES), jnp.float32),
    )(xt, ct)
